# baseline (speedup 1.0000x reference)
import functools

import jax
import jax.numpy as jnp
from jax import lax
from jax.experimental import pallas as pl
from jax.experimental.pallas import tpu as pltpu
from jax.experimental.pallas import tpu_sc as plsc

_NF = 26
_B = 4096
_V = 100001
_D = 32
_HID = [1024, 512, 256, 128]
_TOT = _NF * _D
_EPS = 1e-5

_NC = 2
_NS = 16
_NW = _NC * _NS
_ROWS = _NF * _B
_CHUNK = 128
_NCHUNKS = _ROWS // _CHUNK
_CPW = _NCHUNKS // _NW


def _sc_gather_body(table, idxt, out, idx_v, rows_v, sem):
    wid = lax.axis_index("s") * _NC + lax.axis_index("c")
    cbase = wid * _CPW
    pltpu.sync_copy(idxt.at[wid], idx_v)

    def add_offsets(c, carry):
        r0 = (cbase + c) * _CHUNK
        for k in range(_CHUNK // 16):
            rp = lax.iota(jnp.int32, 16) + (r0 + k * 16)
            fld = lax.rem(rp, _NF)
            sl = pl.ds(k * 16, 16)
            idx_v[c, sl] = idx_v[c, sl] + fld * _V
        return carry

    lax.fori_loop(0, _CPW, add_offsets, 0)

    def gather_one(c, carry):
        pltpu.async_copy(table.at[idx_v.at[c]], rows_v.at[c], sem).wait()
        return carry

    lax.fori_loop(0, _CPW, gather_one, 0)

    pltpu.sync_copy(rows_v, out.at[pl.ds(cbase, _CPW)])


@functools.partial(
    pl.kernel,
    out_type=jax.ShapeDtypeStruct((_NCHUNKS, _CHUNK, _D), jnp.float32),
    mesh=plsc.VectorSubcoreMesh(
        core_axis_name="c", subcore_axis_name="s",
        num_cores=_NC, num_subcores=_NS,
    ),
    scratch_types=[
        pltpu.VMEM((_CPW, _CHUNK), jnp.int32),
        pltpu.VMEM((_CPW, _CHUNK, _D), jnp.float32),
        pltpu.SemaphoreType.DMA,
    ],
    compiler_params=pltpu.CompilerParams(use_tc_tiling_on_sc=False),
)
def _sc_gather(table, idxt, out, idx_v, rows_v, sem):
    _sc_gather_body(table, idxt, out, idx_v, rows_v, sem)


_BLK = 512


def _sigmoid(x):
    return 1.0 / (1.0 + jnp.exp(-x))


def _tc_dense_body(xu_ref, sew1_ref, seb1_ref, sew2_ref, seb2_ref,
                   w0_ref, b0_ref, w1_ref, b1_ref, w2_ref, b2_ref,
                   w3_ref, b3_ref, wo_ref, bo_ref, out_ref):
    xu = xu_ref[...]

    r = lax.broadcasted_iota(jnp.int32, (_TOT, _NF), 0) // _D
    c = lax.broadcasted_iota(jnp.int32, (_TOT, _NF), 1)
    m = jnp.where(r == c, 1.0 / _D, 0.0)
    sq = jnp.dot(xu, m, preferred_element_type=jnp.float32)

    h = jnp.maximum(
        jnp.dot(sq, sew1_ref[...], preferred_element_type=jnp.float32)
        + seb1_ref[...], 0.0)
    wse = _sigmoid(
        jnp.dot(h, sew2_ref[...], preferred_element_type=jnp.float32)
        + seb2_ref[...])

    r2 = lax.broadcasted_iota(jnp.int32, (_NF, _TOT), 0)
    c2 = lax.broadcasted_iota(jnp.int32, (_NF, _TOT), 1) // _D
    e = jnp.where(r2 == c2, 1.0, 0.0)
    x = xu * jnp.dot(wse, e, preferred_element_type=jnp.float32)

    for w_ref, b_ref in ((w0_ref, b0_ref), (w1_ref, b1_ref),
                         (w2_ref, b2_ref), (w3_ref, b3_ref)):
        x = jnp.maximum(
            jnp.dot(x, w_ref[...], preferred_element_type=jnp.float32)
            + b_ref[...], 0.0)

    out_ref[...] = (jnp.dot(x, wo_ref[...], preferred_element_type=jnp.float32)
                    + bo_ref[...])


def _full_spec(shape):
    return pl.BlockSpec(shape, lambda i: tuple(0 for _ in shape))


def _tc_dense(xu, sew1, seb1, sew2, seb2, ws, wo, bo):
    args = [xu, sew1, seb1, sew2, seb2]
    in_specs = [pl.BlockSpec((_BLK, _TOT), lambda i: (i, 0)),
                _full_spec(sew1.shape), _full_spec(seb1.shape),
                _full_spec(sew2.shape), _full_spec(seb2.shape)]
    for w, b in ws:
        args += [w, b]
        in_specs += [_full_spec(w.shape), _full_spec(b.shape)]
    args += [wo, bo]
    in_specs += [_full_spec(wo.shape), _full_spec(bo.shape)]
    return pl.pallas_call(
        _tc_dense_body,
        grid=(_B // _BLK,),
        in_specs=in_specs,
        out_specs=pl.BlockSpec((_BLK, 1), lambda i: (i, 0)),
        out_shape=jax.ShapeDtypeStruct((_B, 1), jnp.float32),
    )(*args)


def kernel(f0, f1, f2, f3, f4, f5, f6, f7, f8, f9, f10, f11, f12, f13, f14,
           f15, f16, f17, f18, f19, f20, f21, f22, f23, f24, f25,
           emb, se_w1, se_b1, se_w2, se_b2,
           w0, b0, g0, be0, w1, b1, g1, be1, w2, b2, g2, be2,
           w3, b3, g3, be3, wo, bo):
    fs = jnp.stack([f0, f1, f2, f3, f4, f5, f6, f7, f8, f9, f10, f11, f12,
                    f13, f14, f15, f16, f17, f18, f19, f20, f21, f22, f23,
                    f24, f25], axis=1)
    idxt = fs.reshape(_NW, _CPW, _CHUNK)
    table = emb.reshape(_NF * _V, _D)

    xu = _sc_gather(table, idxt)
    xu = xu.reshape(_B, _TOT)

    s = 1.0 / jnp.sqrt(jnp.float32(1.0 + _EPS))
    ws = []
    for w, b, g, be in ((w0, b0, g0, be0), (w1, b1, g1, be1),
                        (w2, b2, g2, be2), (w3, b3, g3, be3)):
        gs = g * s
        ws.append((w * gs[None, :], (b * gs + be)[None, :]))

    out = _tc_dense(xu, se_w1, se_b1[None, :], se_w2, se_b2[None, :],
                    ws, wo, bo[None, :])
    return out[:, 0]


# speedup vs baseline: 1.9540x; 1.9540x over previous
import functools

import jax
import jax.numpy as jnp
from jax import lax
from jax.experimental import pallas as pl
from jax.experimental.pallas import tpu as pltpu
from jax.experimental.pallas import tpu_sc as plsc

_NF = 26
_B = 4096
_V = 100001
_D = 32
_TOT = _NF * _D
_EPS = 1e-5

_VPK = 25024
_VG = 17
_VROWS = _VPK // _VG
_VSPAN = _VROWS * 4

_NC = 2
_NS = 16
_NW = _NC * _NS
_CHUNK = 128
_NCHUNKS = _NF * _B // _CHUNK
_CPW = _NCHUNKS // _NW
_BPC = _B // _CHUNK


def _repack_body(src_ref, out_ref):
    x = src_ref[0]
    xt = jnp.transpose(x.reshape(_D, _VROWS, 4), (1, 2, 0))
    out_ref[0] = xt.reshape(_VROWS, 4 * _D)


def _repack(tableT):
    return pl.pallas_call(
        _repack_body,
        grid=(_NF, _VG),
        in_specs=[pl.BlockSpec((1, _D, _VSPAN), lambda f, h: (f, 0, h))],
        out_specs=pl.BlockSpec((1, _VROWS, 4 * _D), lambda f, h: (f, h, 0)),
        out_shape=jax.ShapeDtypeStruct((_NF, _VPK, 4 * _D), jnp.float32),
    )(tableT)


def _sc_gather_body(tpk, idxg, out, idx_v, pidx_v, packed_v, slab_v, sem):
    wid = lax.axis_index("s") * _NC + lax.axis_index("c")
    pltpu.sync_copy(idxg.at[wid], idx_v)

    def chunk_body(c, carry):
        g = wid * _CPW + c
        f = g // _BPC
        bb = lax.rem(g, _BPC)
        for jg in range(_CHUNK // 16):
            sl = pl.ds(jg * 16, 16)
            pidx_v[sl] = f * _VPK + lax.shift_right_logical(idx_v[c, sl], 2)
        pltpu.async_copy(tpk.at[pidx_v], packed_v, sem).wait()
        lanes = lax.iota(jnp.int32, 16)
        for jg in range(_CHUNK // 16):
            vec = idx_v[c, pl.ds(jg * 16, 16)]
            for k in range(16):
                j = jg * 16 + k
                o = lax.rem(vec[k], 4) * _D
                lo = packed_v[j, pl.ds(o, 16)]
                hi = packed_v[j, pl.ds(o + 16, 16)]
                plsc.store_scatter(slab_v, [lanes, jnp.full((16,), j,
                                                            jnp.int32)], lo)
                plsc.store_scatter(slab_v, [lanes + 16, jnp.full((16,), j,
                                                                 jnp.int32)],
                                   hi)
        pltpu.sync_copy(slab_v, out.at[f, bb])
        return carry

    lax.fori_loop(0, _CPW, chunk_body, 0)


@functools.partial(
    pl.kernel,
    out_type=jax.ShapeDtypeStruct((_NF, _BPC, _D, _CHUNK), jnp.float32),
    mesh=plsc.VectorSubcoreMesh(
        core_axis_name="c", subcore_axis_name="s",
        num_cores=_NC, num_subcores=_NS,
    ),
    scratch_types=[
        pltpu.VMEM((_CPW, _CHUNK), jnp.int32),
        pltpu.VMEM((_CHUNK,), jnp.int32),
        pltpu.VMEM((_CHUNK, 4 * _D), jnp.float32),
        pltpu.VMEM((_D, _CHUNK), jnp.float32),
        pltpu.SemaphoreType.DMA,
    ],
    compiler_params=pltpu.CompilerParams(use_tc_tiling_on_sc=True,
                                         needs_layout_passes=False),
)
def _sc_gather(tpk, idxg, out, idx_v, pidx_v, packed_v, slab_v, sem):
    _sc_gather_body(tpk, idxg, out, idx_v, pidx_v, packed_v, slab_v, sem)


_BLK = 512


def _sigmoid(x):
    return 1.0 / (1.0 + jnp.exp(-x))


def _tc_dense_body(xu_ref, sew1t_ref, seb1c_ref, sew2t_ref, seb2c_ref,
                   w0t_ref, b0c_ref, w1t_ref, b1c_ref, w2t_ref, b2c_ref,
                   w3t_ref, b3c_ref, wot_ref, bo_ref, out_ref):
    x4 = xu_ref[...]
    xu = jnp.transpose(x4, (0, 2, 1, 3)).reshape(_TOT, _BLK)

    ri = lax.broadcasted_iota(jnp.int32, (_NF, _TOT), 0)
    ci = lax.broadcasted_iota(jnp.int32, (_NF, _TOT), 1) // _D
    mT = jnp.where(ri == ci, 1.0 / _D, 0.0)
    sqT = jnp.dot(mT, xu, preferred_element_type=jnp.float32)

    hT = jnp.maximum(
        jnp.dot(sew1t_ref[...], sqT, preferred_element_type=jnp.float32)
        + seb1c_ref[...], 0.0)
    wseT = _sigmoid(
        jnp.dot(sew2t_ref[...], hT, preferred_element_type=jnp.float32)
        + seb2c_ref[...])

    r2 = lax.broadcasted_iota(jnp.int32, (_TOT, _NF), 0) // _D
    c2 = lax.broadcasted_iota(jnp.int32, (_TOT, _NF), 1)
    eT = jnp.where(r2 == c2, 1.0, 0.0)
    x = xu * jnp.dot(eT, wseT, preferred_element_type=jnp.float32)

    for wt_ref, bc_ref in ((w0t_ref, b0c_ref), (w1t_ref, b1c_ref),
                           (w2t_ref, b2c_ref), (w3t_ref, b3c_ref)):
        x = jnp.maximum(
            jnp.dot(wt_ref[...], x, preferred_element_type=jnp.float32)
            + bc_ref[...], 0.0)

    out_ref[...] = (jnp.dot(wot_ref[...], x,
                            preferred_element_type=jnp.float32) + bo_ref[...])


def _full_spec(shape):
    return pl.BlockSpec(shape, lambda i: tuple(0 for _ in shape))


def _tc_dense(xu4, sew1t, seb1c, sew2t, seb2c, ws, wot, bo):
    args = [xu4, sew1t, seb1c, sew2t, seb2c]
    in_specs = [pl.BlockSpec((_NF, _BLK // _CHUNK, _D, _CHUNK),
                             lambda i: (0, i, 0, 0)),
                _full_spec(sew1t.shape), _full_spec(seb1c.shape),
                _full_spec(sew2t.shape), _full_spec(seb2c.shape)]
    for wt, bc in ws:
        args += [wt, bc]
        in_specs += [_full_spec(wt.shape), _full_spec(bc.shape)]
    args += [wot, bo]
    in_specs += [_full_spec(wot.shape), _full_spec(bo.shape)]
    return pl.pallas_call(
        _tc_dense_body,
        grid=(_B // _BLK,),
        in_specs=in_specs,
        out_specs=pl.BlockSpec((1, _BLK), lambda i: (0, i)),
        out_shape=jax.ShapeDtypeStruct((1, _B), jnp.float32),
    )(*args)


def kernel(f0, f1, f2, f3, f4, f5, f6, f7, f8, f9, f10, f11, f12, f13, f14,
           f15, f16, f17, f18, f19, f20, f21, f22, f23, f24, f25,
           emb, se_w1, se_b1, se_w2, se_b2,
           w0, b0, g0, be0, w1, b1, g1, be1, w2, b2, g2, be2,
           w3, b3, g3, be3, wo, bo):
    fs = jnp.stack([f0, f1, f2, f3, f4, f5, f6, f7, f8, f9, f10, f11, f12,
                    f13, f14, f15, f16, f17, f18, f19, f20, f21, f22, f23,
                    f24, f25], axis=0)
    idxg = fs.reshape(_NW, _CPW, _CHUNK)
    tableT = jnp.transpose(emb, (0, 2, 1))

    tpk = _repack(tableT)
    tpk2 = tpk.reshape(_NF * _VPK, 4 * _D)

    xu4 = _sc_gather(tpk2, idxg)

    s = 1.0 / jnp.sqrt(jnp.float32(1.0 + _EPS))
    ws = []
    for w, b, g, be in ((w0, b0, g0, be0), (w1, b1, g1, be1),
                        (w2, b2, g2, be2), (w3, b3, g3, be3)):
        gs = g * s
        ws.append(((w * gs[None, :]).T, (b * gs + be)[:, None]))

    out = _tc_dense(xu4, se_w1.T, se_b1[:, None], se_w2.T, se_b2[:, None],
                    ws, wo.T, bo[:, None])
    return out[0]


# speedup vs baseline: 31.7236x; 16.2355x over previous
import functools

import jax
import jax.numpy as jnp
from jax import lax
from jax.experimental import pallas as pl
from jax.experimental.pallas import tpu as pltpu
from jax.experimental.pallas import tpu_sc as plsc

_NF = 26
_B = 4096
_V = 100001
_D = 32
_TOT = _NF * _D
_EPS = 1e-5

_VPK = 25088
_VG = 4
_VROWS = _VPK // _VG

_NC = 2
_NS = 16
_NW = _NC * _NS
_CHUNK = 128
_NCHUNKS = _NF * _B // _CHUNK
_CPW = _NCHUNKS // _NW
_BPC = _B // _CHUNK


def _repack_body(s0_ref, s1_ref, s2_ref, s3_ref, out_ref):
    ri = lax.broadcasted_iota(jnp.int32, (4 * _D, 4 * _D), 0)
    ci = lax.broadcasted_iota(jnp.int32, (4 * _D, 4 * _D), 1)
    eye = jnp.where(ri == ci, 1.0, 0.0)
    x4 = jnp.concatenate([s0_ref[0], s1_ref[0], s2_ref[0], s3_ref[0]],
                         axis=0)
    out_ref[0] = lax.dot_general(x4, eye, (((0,), (0,)), ((), ())),
                                 preferred_element_type=jnp.float32)


def _repack(tableT):
    def spec(s):
        return pl.BlockSpec((1, _D, _VROWS),
                            lambda f, h, s=s: (f, 0, s * _VG + h))
    return pl.pallas_call(
        _repack_body,
        grid=(_NF, _VG),
        in_specs=[spec(0), spec(1), spec(2), spec(3)],
        out_specs=pl.BlockSpec((1, _VROWS, 4 * _D), lambda f, h: (f, h, 0)),
        out_shape=jax.ShapeDtypeStruct((_NF, _VPK, 4 * _D), jnp.float32),
        compiler_params=pltpu.CompilerParams(
            fuse_transposed_lhs_in_matmul=True),
    )(tableT, tableT, tableT, tableT)


def _sc_gather_body(tpk, idxg, out, idx_v, pidx_v, packed_v, slab_v, sem):
    wid = lax.axis_index("s") * _NC + lax.axis_index("c")
    pltpu.sync_copy(idxg.at[wid], idx_v)

    def chunk_body(c, carry):
        g = wid * _CPW + c
        f = g // _BPC
        bb = lax.rem(g, _BPC)
        for jg in range(_CHUNK // 16):
            sl = pl.ds(jg * 16, 16)
            pidx_v[sl] = f * _VPK + lax.rem(idx_v[c, sl], _VPK)
        pltpu.async_copy(tpk.at[pidx_v], packed_v, sem).wait()
        lanes = lax.iota(jnp.int32, 16)
        for jg in range(_CHUNK // 16):
            vec = idx_v[c, pl.ds(jg * 16, 16)]
            for k in range(16):
                j = jg * 16 + k
                o = (vec[k] // _VPK) * _D
                lo = packed_v[j, pl.ds(o, 16)]
                hi = packed_v[j, pl.ds(o + 16, 16)]
                plsc.store_scatter(slab_v, [lanes, jnp.full((16,), j,
                                                            jnp.int32)], lo)
                plsc.store_scatter(slab_v, [lanes + 16, jnp.full((16,), j,
                                                                 jnp.int32)],
                                   hi)
        pltpu.sync_copy(slab_v, out.at[f, bb])
        return carry

    lax.fori_loop(0, _CPW, chunk_body, 0)


@functools.partial(
    pl.kernel,
    out_type=jax.ShapeDtypeStruct((_NF, _BPC, _D, _CHUNK), jnp.float32),
    mesh=plsc.VectorSubcoreMesh(
        core_axis_name="c", subcore_axis_name="s",
        num_cores=_NC, num_subcores=_NS,
    ),
    scratch_types=[
        pltpu.VMEM((_CPW, _CHUNK), jnp.int32),
        pltpu.VMEM((_CHUNK,), jnp.int32),
        pltpu.VMEM((_CHUNK, 4 * _D), jnp.float32),
        pltpu.VMEM((_D, _CHUNK), jnp.float32),
        pltpu.SemaphoreType.DMA,
    ],
    compiler_params=pltpu.CompilerParams(use_tc_tiling_on_sc=True,
                                         needs_layout_passes=False),
)
def _sc_gather(tpk, idxg, out, idx_v, pidx_v, packed_v, slab_v, sem):
    _sc_gather_body(tpk, idxg, out, idx_v, pidx_v, packed_v, slab_v, sem)


_BLK = 512


def _sigmoid(x):
    return 1.0 / (1.0 + jnp.exp(-x))


def _tc_dense_body(xu_ref, sew1t_ref, seb1c_ref, sew2t_ref, seb2c_ref,
                   w0t_ref, b0c_ref, w1t_ref, b1c_ref, w2t_ref, b2c_ref,
                   w3t_ref, b3c_ref, wot_ref, bo_ref, out_ref):
    x4 = xu_ref[...]
    xu = jnp.transpose(x4, (0, 2, 1, 3)).reshape(_TOT, _BLK)

    ri = lax.broadcasted_iota(jnp.int32, (_NF, _TOT), 0)
    ci = lax.broadcasted_iota(jnp.int32, (_NF, _TOT), 1) // _D
    mT = jnp.where(ri == ci, 1.0 / _D, 0.0)
    sqT = jnp.dot(mT, xu, preferred_element_type=jnp.float32)

    hT = jnp.maximum(
        jnp.dot(sew1t_ref[...], sqT, preferred_element_type=jnp.float32)
        + seb1c_ref[...], 0.0)
    wseT = _sigmoid(
        jnp.dot(sew2t_ref[...], hT, preferred_element_type=jnp.float32)
        + seb2c_ref[...])

    r2 = lax.broadcasted_iota(jnp.int32, (_TOT, _NF), 0) // _D
    c2 = lax.broadcasted_iota(jnp.int32, (_TOT, _NF), 1)
    eT = jnp.where(r2 == c2, 1.0, 0.0)
    x = xu * jnp.dot(eT, wseT, preferred_element_type=jnp.float32)

    for wt_ref, bc_ref in ((w0t_ref, b0c_ref), (w1t_ref, b1c_ref),
                           (w2t_ref, b2c_ref), (w3t_ref, b3c_ref)):
        x = jnp.maximum(
            jnp.dot(wt_ref[...], x, preferred_element_type=jnp.float32)
            + bc_ref[...], 0.0)

    out_ref[...] = (jnp.dot(wot_ref[...], x,
                            preferred_element_type=jnp.float32) + bo_ref[...])


def _full_spec(shape):
    return pl.BlockSpec(shape, lambda i: tuple(0 for _ in shape))


def _tc_dense(xu4, sew1t, seb1c, sew2t, seb2c, ws, wot, bo):
    args = [xu4, sew1t, seb1c, sew2t, seb2c]
    in_specs = [pl.BlockSpec((_NF, _BLK // _CHUNK, _D, _CHUNK),
                             lambda i: (0, i, 0, 0)),
                _full_spec(sew1t.shape), _full_spec(seb1c.shape),
                _full_spec(sew2t.shape), _full_spec(seb2c.shape)]
    for wt, bc in ws:
        args += [wt, bc]
        in_specs += [_full_spec(wt.shape), _full_spec(bc.shape)]
    args += [wot, bo]
    in_specs += [_full_spec(wot.shape), _full_spec(bo.shape)]
    return pl.pallas_call(
        _tc_dense_body,
        grid=(_B // _BLK,),
        in_specs=in_specs,
        out_specs=pl.BlockSpec((1, _BLK), lambda i: (0, i)),
        out_shape=jax.ShapeDtypeStruct((1, _B), jnp.float32),
    )(*args)


def kernel(f0, f1, f2, f3, f4, f5, f6, f7, f8, f9, f10, f11, f12, f13, f14,
           f15, f16, f17, f18, f19, f20, f21, f22, f23, f24, f25,
           emb, se_w1, se_b1, se_w2, se_b2,
           w0, b0, g0, be0, w1, b1, g1, be1, w2, b2, g2, be2,
           w3, b3, g3, be3, wo, bo):
    fs = jnp.stack([f0, f1, f2, f3, f4, f5, f6, f7, f8, f9, f10, f11, f12,
                    f13, f14, f15, f16, f17, f18, f19, f20, f21, f22, f23,
                    f24, f25], axis=0)
    idxg = fs.reshape(_NW, _CPW, _CHUNK)
    tableT = jnp.transpose(emb, (0, 2, 1))

    tpk = _repack(tableT)
    tpk2 = tpk.reshape(_NF * _VPK, 4 * _D)

    xu4 = _sc_gather(tpk2, idxg)

    s = 1.0 / jnp.sqrt(jnp.float32(1.0 + _EPS))
    ws = []
    for w, b, g, be in ((w0, b0, g0, be0), (w1, b1, g1, be1),
                        (w2, b2, g2, be2), (w3, b3, g3, be3)):
        gs = g * s
        ws.append(((w * gs[None, :]).T, (b * gs + be)[:, None]))

    out = _tc_dense(xu4, se_w1.T, se_b1[:, None], se_w2.T, se_b2[:, None],
                    ws, wo.T, bo[:, None])
    return out[0]


# speedup vs baseline: 40.5233x; 1.2774x over previous
import functools

import jax
import jax.numpy as jnp
from jax import lax
from jax.experimental import pallas as pl
from jax.experimental.pallas import tpu as pltpu
from jax.experimental.pallas import tpu_sc as plsc

_NF = 26
_B = 4096
_V = 100001
_D = 32
_HD = _D // 2
_TOT = _NF * _D
_EPS = 1e-5

_VPK = 12544
_NSLOT = 8
_VG = 2
_VROWS = _VPK // _VG

_NC = 2
_NS = 16
_NW = _NC * _NS
_CHUNK = 128
_NCHUNKS = _NF * _B // _CHUNK
_CPW = _NCHUNKS // _NW
_BPC = _B // _CHUNK


def _repack_body(*refs):
    out_ref = refs[-1]
    ki = lax.broadcasted_iota(jnp.int32, (_NSLOT * _D, 128), 0)
    ci = lax.broadcasted_iota(jnp.int32, (_NSLOT * _D, 128), 1)
    src_even = (ci // _HD) * _D + 2 * lax.rem(ci, _HD)
    sel_e = jnp.where(ki == src_even, 1.0, 0.0).astype(jnp.bfloat16)
    sel_o = jnp.where(ki == src_even + 1, 1.0, 0.0).astype(jnp.bfloat16)

    x8 = jnp.concatenate([r[0] for r in refs[:-1]],
                         axis=0).astype(jnp.bfloat16)
    dims = (((0,), (0,)), ((), ()))
    t_e = lax.dot_general(x8, sel_e, dims,
                          preferred_element_type=jnp.float32)
    t_o = lax.dot_general(x8, sel_o, dims,
                          preferred_element_type=jnp.float32)
    be = lax.bitcast_convert_type(t_e, jnp.int32)
    bo = lax.bitcast_convert_type(t_o, jnp.int32)
    out_ref[0] = jnp.bitwise_or(
        lax.shift_right_logical(be, 16),
        jnp.bitwise_and(bo, jnp.int32(-65536)))


def _repack(tableT):
    def spec(s):
        return pl.BlockSpec((1, _D, _VROWS),
                            lambda f, h, s=s: (f, 0, s * _VG + h))
    return pl.pallas_call(
        _repack_body,
        grid=(_NF, _VG),
        in_specs=[spec(s) for s in range(_NSLOT)],
        out_specs=pl.BlockSpec((1, _VROWS, 128), lambda f, h: (f, h, 0)),
        out_shape=jax.ShapeDtypeStruct((_NF, _VPK, 128), jnp.int32),
        compiler_params=pltpu.CompilerParams(
            fuse_transposed_lhs_in_matmul=True),
    )(*([tableT] * _NSLOT))


def _sc_gather_body(tpk, idxg, out, idx_v, pidx_v, pk_a, pk_b, slab_v, sem_a,
                    sem_b):
    wid = lax.axis_index("s") * _NC + lax.axis_index("c")
    pltpu.sync_copy(idxg.at[wid], idx_v)

    def pidx_chunk(c, carry):
        g = wid * _CPW + c
        f = g // _BPC
        for jg in range(_CHUNK // 16):
            sl = pl.ds(jg * 16, 16)
            pidx_v[c, sl] = f * _VPK + lax.rem(idx_v[c, sl], _VPK)
        return carry

    lax.fori_loop(0, _CPW, pidx_chunk, 0)

    def fire(c, pk, sem):
        pltpu.async_copy(tpk.at[pidx_v.at[c]], pk, sem)

    def wait(pk, sem):
        pltpu.make_async_copy(tpk.at[pl.ds(0, _CHUNK)], pk, sem).wait()

    def extract(c, pk):
        g = wid * _CPW + c
        f = g // _BPC
        bb = lax.rem(g, _BPC)
        lanes = lax.iota(jnp.int32, 16)
        for jg in range(_CHUNK // 16):
            vec = idx_v[c, pl.ds(jg * 16, 16)]
            for k in range(16):
                j = jg * 16 + k
                o = (vec[k] // _VPK) * _HD
                plsc.store_scatter(slab_v,
                                   [lanes, jnp.full((16,), j, jnp.int32)],
                                   pk[j, pl.ds(o, 16)])
        pltpu.sync_copy(slab_v, out.at[f, bb])

    fire(0, pk_a, sem_a)

    def pipe(cc, carry):
        c0 = cc * 2
        fire(c0 + 1, pk_b, sem_b)
        wait(pk_a, sem_a)
        extract(c0, pk_a)

        @pl.when(cc < _CPW // 2 - 1)
        def _():
            fire(c0 + 2, pk_a, sem_a)

        wait(pk_b, sem_b)
        extract(c0 + 1, pk_b)
        return carry

    lax.fori_loop(0, _CPW // 2, pipe, 0)


@functools.partial(
    pl.kernel,
    out_type=jax.ShapeDtypeStruct((_NF, _BPC, _HD, _CHUNK), jnp.int32),
    mesh=plsc.VectorSubcoreMesh(
        core_axis_name="c", subcore_axis_name="s",
        num_cores=_NC, num_subcores=_NS,
    ),
    scratch_types=[
        pltpu.VMEM((_CPW, _CHUNK), jnp.int32),
        pltpu.VMEM((_CPW, _CHUNK), jnp.int32),
        pltpu.VMEM((_CHUNK, 128), jnp.int32),
        pltpu.VMEM((_CHUNK, 128), jnp.int32),
        pltpu.VMEM((_HD, _CHUNK), jnp.int32),
        pltpu.SemaphoreType.DMA,
        pltpu.SemaphoreType.DMA,
    ],
    compiler_params=pltpu.CompilerParams(use_tc_tiling_on_sc=True,
                                         needs_layout_passes=False),
)
def _sc_gather(tpk, idxg, out, idx_v, pidx_v, pk_a, pk_b, slab_v, sem_a,
               sem_b):
    _sc_gather_body(tpk, idxg, out, idx_v, pidx_v, pk_a, pk_b, slab_v,
                    sem_a, sem_b)


_BLK = 512
_HTOT = _NF * _HD


def _sigmoid(x):
    return 1.0 / (1.0 + jnp.exp(-x))


def _tc_dense_body(xi_ref, sew1t_ref, seb1c_ref, sew2t_ref, seb2c_ref,
                   w0te_ref, w0to_ref, b0c_ref, w1t_ref, b1c_ref,
                   w2t_ref, b2c_ref, w3t_ref, b3c_ref, wot_ref, bo_ref,
                   out_ref):
    w4 = xi_ref[...]
    wi = jnp.transpose(w4, (0, 2, 1, 3)).reshape(_HTOT, _BLK)
    xe = lax.bitcast_convert_type(lax.shift_left(wi, 16), jnp.float32)
    xo = lax.bitcast_convert_type(
        jnp.bitwise_and(wi, jnp.int32(-65536)), jnp.float32)

    ri = lax.broadcasted_iota(jnp.int32, (_NF, _HTOT), 0)
    ci = lax.broadcasted_iota(jnp.int32, (_NF, _HTOT), 1) // _HD
    mT = jnp.where(ri == ci, 1.0 / _D, 0.0)
    sqT = (jnp.dot(mT, xe, preferred_element_type=jnp.float32)
           + jnp.dot(mT, xo, preferred_element_type=jnp.float32))

    hT = jnp.maximum(
        jnp.dot(sew1t_ref[...], sqT, preferred_element_type=jnp.float32)
        + seb1c_ref[...], 0.0)
    wseT = _sigmoid(
        jnp.dot(sew2t_ref[...], hT, preferred_element_type=jnp.float32)
        + seb2c_ref[...])

    r2 = lax.broadcasted_iota(jnp.int32, (_HTOT, _NF), 0) // _HD
    c2 = lax.broadcasted_iota(jnp.int32, (_HTOT, _NF), 1)
    eT = jnp.where(r2 == c2, 1.0, 0.0)
    scale = jnp.dot(eT, wseT, preferred_element_type=jnp.float32)
    xe = xe * scale
    xo = xo * scale

    x = jnp.maximum(
        jnp.dot(w0te_ref[...], xe, preferred_element_type=jnp.float32)
        + jnp.dot(w0to_ref[...], xo, preferred_element_type=jnp.float32)
        + b0c_ref[...], 0.0)
    for wt_ref, bc_ref in ((w1t_ref, b1c_ref), (w2t_ref, b2c_ref),
                           (w3t_ref, b3c_ref)):
        x = jnp.maximum(
            jnp.dot(wt_ref[...], x, preferred_element_type=jnp.float32)
            + bc_ref[...], 0.0)

    out_ref[...] = (jnp.dot(wot_ref[...], x,
                            preferred_element_type=jnp.float32) + bo_ref[...])


def _full_spec(shape):
    return pl.BlockSpec(shape, lambda i: tuple(0 for _ in shape))


def _tc_dense(xi4, sew1t, seb1c, sew2t, seb2c, w0te, w0to, b0c, ws, wot, bo):
    args = [xi4, sew1t, seb1c, sew2t, seb2c, w0te, w0to, b0c]
    in_specs = [pl.BlockSpec((_NF, _BLK // _CHUNK, _HD, _CHUNK),
                             lambda i: (0, i, 0, 0)),
                _full_spec(sew1t.shape), _full_spec(seb1c.shape),
                _full_spec(sew2t.shape), _full_spec(seb2c.shape),
                _full_spec(w0te.shape), _full_spec(w0to.shape),
                _full_spec(b0c.shape)]
    for wt, bc in ws:
        args += [wt, bc]
        in_specs += [_full_spec(wt.shape), _full_spec(bc.shape)]
    args += [wot, bo]
    in_specs += [_full_spec(wot.shape), _full_spec(bo.shape)]
    return pl.pallas_call(
        _tc_dense_body,
        grid=(_B // _BLK,),
        in_specs=in_specs,
        out_specs=pl.BlockSpec((1, _BLK), lambda i: (0, i)),
        out_shape=jax.ShapeDtypeStruct((1, _B), jnp.float32),
    )(*args)


def kernel(f0, f1, f2, f3, f4, f5, f6, f7, f8, f9, f10, f11, f12, f13, f14,
           f15, f16, f17, f18, f19, f20, f21, f22, f23, f24, f25,
           emb, se_w1, se_b1, se_w2, se_b2,
           w0, b0, g0, be0, w1, b1, g1, be1, w2, b2, g2, be2,
           w3, b3, g3, be3, wo, bo):
    fs = jnp.stack([f0, f1, f2, f3, f4, f5, f6, f7, f8, f9, f10, f11, f12,
                    f13, f14, f15, f16, f17, f18, f19, f20, f21, f22, f23,
                    f24, f25], axis=0)
    idxg = fs.reshape(_NW, _CPW, _CHUNK)
    tableT = jnp.transpose(emb, (0, 2, 1))

    tpk = _repack(tableT)
    tpk2 = tpk.reshape(_NF * _VPK, 128)

    xi4 = _sc_gather(tpk2, idxg)

    s = 1.0 / jnp.sqrt(jnp.float32(1.0 + _EPS))
    folded = []
    for w, b, g, be in ((w0, b0, g0, be0), (w1, b1, g1, be1),
                        (w2, b2, g2, be2), (w3, b3, g3, be3)):
        gs = g * s
        folded.append((w * gs[None, :], (b * gs + be)[:, None]))
    w0f, b0c = folded[0]
    w0r = w0f.reshape(_NF, _D, -1)
    w0te = w0r[:, 0::2, :].reshape(_HTOT, -1).T
    w0to = w0r[:, 1::2, :].reshape(_HTOT, -1).T
    ws = [(wt.T, bc) for wt, bc in folded[1:]]

    out = _tc_dense(xi4, se_w1.T, se_b1[:, None], se_w2.T, se_b2[:, None],
                    w0te, w0to, b0c, ws, wo.T, bo[:, None])
    return out[0]


# speedup vs baseline: 42.4981x; 1.0487x over previous
import functools

import jax
import jax.numpy as jnp
from jax import lax
from jax.experimental import pallas as pl
from jax.experimental.pallas import tpu as pltpu
from jax.experimental.pallas import tpu_sc as plsc

_NF = 26
_B = 4096
_V = 100001
_D = 32
_HD = _D // 2
_TOT = _NF * _D
_EPS = 1e-5

_VPK = 12544
_NSLOT = 8
_VG = 2
_VROWS = _VPK // _VG

_NC = 2
_NS = 16
_NW = _NC * _NS
_CHUNK = 128
_NCHUNKS = _NF * _B // _CHUNK
_CPW = _NCHUNKS // _NW
_BPC = _B // _CHUNK


def _repack_body(*refs):
    out_ref = refs[-1]
    ki = lax.broadcasted_iota(jnp.int32, (_NSLOT * _D, 128), 0)
    ci = lax.broadcasted_iota(jnp.int32, (_NSLOT * _D, 128), 1)
    src_even = (ci // _HD) * _D + 2 * lax.rem(ci, _HD)
    sel_e = jnp.where(ki == src_even, 1.0, 0.0).astype(jnp.bfloat16)
    sel_o = jnp.where(ki == src_even + 1, 1.0, 0.0).astype(jnp.bfloat16)

    x8 = jnp.concatenate([r[0] for r in refs[:-1]],
                         axis=0).astype(jnp.bfloat16)
    dims = (((0,), (0,)), ((), ()))
    t_e = lax.dot_general(x8, sel_e, dims,
                          preferred_element_type=jnp.float32)
    t_o = lax.dot_general(x8, sel_o, dims,
                          preferred_element_type=jnp.float32)
    be = lax.bitcast_convert_type(t_e, jnp.int32)
    bo = lax.bitcast_convert_type(t_o, jnp.int32)
    out_ref[0] = jnp.bitwise_or(
        lax.shift_right_logical(be, 16),
        jnp.bitwise_and(bo, jnp.int32(-65536)))


def _repack(tableT):
    def spec(s):
        return pl.BlockSpec((1, _D, _VROWS),
                            lambda f, h, s=s: (f, 0, s * _VG + h))
    return pl.pallas_call(
        _repack_body,
        grid=(_NF, _VG),
        in_specs=[spec(s) for s in range(_NSLOT)],
        out_specs=pl.BlockSpec((1, _VROWS, 128), lambda f, h: (f, h, 0)),
        out_shape=jax.ShapeDtypeStruct((_NF, _VPK, 128), jnp.int32),
        compiler_params=pltpu.CompilerParams(
            fuse_transposed_lhs_in_matmul=True),
    )(*([tableT] * _NSLOT))


def _sc_gather_body(tpk, idxg, out, idx_v, pidx_v, pk_a, pk_b, slab_v, sem_a,
                    sem_b):
    wid = lax.axis_index("s") * _NC + lax.axis_index("c")
    pltpu.sync_copy(idxg.at[wid], idx_v)

    def pidx_chunk(c, carry):
        g = wid * _CPW + c
        f = g // _BPC
        for jg in range(_CHUNK // 16):
            sl = pl.ds(c * _CHUNK + jg * 16, 16)
            v = idx_v[sl]
            pidx_v[sl] = ((f * _VPK + lax.rem(v, _VPK)) * _NSLOT
                          + v // _VPK)
        return carry

    lax.fori_loop(0, _CPW, pidx_chunk, 0)

    def fire(c, pk, sem):
        pltpu.async_copy(tpk.at[pidx_v.at[pl.ds(c * _CHUNK, _CHUNK)]], pk,
                         sem)

    def wait(pk, sem):
        pltpu.make_async_copy(tpk.at[pl.ds(0, _CHUNK)], pk, sem).wait()

    def extract(c, pk):
        g = wid * _CPW + c
        f = g // _BPC
        bb = lax.rem(g, _BPC)
        lanes = lax.iota(jnp.int32, 16)
        for jg in range(_CHUNK // 16):
            for k in range(16):
                j = jg * 16 + k
                plsc.store_scatter(slab_v,
                                   [lanes, jnp.full((16,), j, jnp.int32)],
                                   pk[j])
        pltpu.sync_copy(slab_v, out.at[f, bb])

    fire(0, pk_a, sem_a)

    def pipe(cc, carry):
        c0 = cc * 2
        fire(c0 + 1, pk_b, sem_b)
        wait(pk_a, sem_a)
        extract(c0, pk_a)

        @pl.when(cc < _CPW // 2 - 1)
        def _():
            fire(c0 + 2, pk_a, sem_a)

        wait(pk_b, sem_b)
        extract(c0 + 1, pk_b)
        return carry

    lax.fori_loop(0, _CPW // 2, pipe, 0)


@functools.partial(
    pl.kernel,
    out_type=jax.ShapeDtypeStruct((_NF, _BPC, _HD, _CHUNK), jnp.int32),
    mesh=plsc.VectorSubcoreMesh(
        core_axis_name="c", subcore_axis_name="s",
        num_cores=_NC, num_subcores=_NS,
    ),
    scratch_types=[
        pltpu.VMEM((_CPW * _CHUNK,), jnp.int32),
        pltpu.VMEM((_CPW * _CHUNK,), jnp.int32),
        pltpu.VMEM((_CHUNK, _HD), jnp.int32),
        pltpu.VMEM((_CHUNK, _HD), jnp.int32),
        pltpu.VMEM((_HD, _CHUNK), jnp.int32),
        pltpu.SemaphoreType.DMA,
        pltpu.SemaphoreType.DMA,
    ],
    compiler_params=pltpu.CompilerParams(use_tc_tiling_on_sc=False,
                                         needs_layout_passes=False),
)
def _sc_gather(tpk, idxg, out, idx_v, pidx_v, pk_a, pk_b, slab_v, sem_a,
               sem_b):
    _sc_gather_body(tpk, idxg, out, idx_v, pidx_v, pk_a, pk_b, slab_v,
                    sem_a, sem_b)


_BLK = 512
_HTOT = _NF * _HD


def _sigmoid(x):
    return 1.0 / (1.0 + jnp.exp(-x))


def _tc_dense_body(xi_ref, sew1t_ref, seb1c_ref, sew2t_ref, seb2c_ref,
                   w0te_ref, w0to_ref, b0c_ref, w1t_ref, b1c_ref,
                   w2t_ref, b2c_ref, w3t_ref, b3c_ref, wot_ref, bo_ref,
                   out_ref):
    w4 = xi_ref[...]
    wi = jnp.transpose(w4, (0, 2, 1, 3)).reshape(_HTOT, _BLK)
    xe = lax.bitcast_convert_type(lax.shift_left(wi, 16), jnp.float32)
    xo = lax.bitcast_convert_type(
        jnp.bitwise_and(wi, jnp.int32(-65536)), jnp.float32)

    ri = lax.broadcasted_iota(jnp.int32, (_NF, _HTOT), 0)
    ci = lax.broadcasted_iota(jnp.int32, (_NF, _HTOT), 1) // _HD
    mT = jnp.where(ri == ci, 1.0 / _D, 0.0)
    sqT = (jnp.dot(mT, xe, preferred_element_type=jnp.float32)
           + jnp.dot(mT, xo, preferred_element_type=jnp.float32))

    hT = jnp.maximum(
        jnp.dot(sew1t_ref[...], sqT, preferred_element_type=jnp.float32)
        + seb1c_ref[...], 0.0)
    wseT = _sigmoid(
        jnp.dot(sew2t_ref[...], hT, preferred_element_type=jnp.float32)
        + seb2c_ref[...])

    r2 = lax.broadcasted_iota(jnp.int32, (_HTOT, _NF), 0) // _HD
    c2 = lax.broadcasted_iota(jnp.int32, (_HTOT, _NF), 1)
    eT = jnp.where(r2 == c2, 1.0, 0.0)
    scale = jnp.dot(eT, wseT, preferred_element_type=jnp.float32)
    xe = xe * scale
    xo = xo * scale

    x = jnp.maximum(
        jnp.dot(w0te_ref[...], xe, preferred_element_type=jnp.float32)
        + jnp.dot(w0to_ref[...], xo, preferred_element_type=jnp.float32)
        + b0c_ref[...], 0.0)
    for wt_ref, bc_ref in ((w1t_ref, b1c_ref), (w2t_ref, b2c_ref),
                           (w3t_ref, b3c_ref)):
        x = jnp.maximum(
            jnp.dot(wt_ref[...], x, preferred_element_type=jnp.float32)
            + bc_ref[...], 0.0)

    out_ref[...] = (jnp.dot(wot_ref[...], x,
                            preferred_element_type=jnp.float32) + bo_ref[...])


def _full_spec(shape):
    return pl.BlockSpec(shape, lambda i: tuple(0 for _ in shape))


def _tc_dense(xi4, sew1t, seb1c, sew2t, seb2c, w0te, w0to, b0c, ws, wot, bo):
    args = [xi4, sew1t, seb1c, sew2t, seb2c, w0te, w0to, b0c]
    in_specs = [pl.BlockSpec((_NF, _BLK // _CHUNK, _HD, _CHUNK),
                             lambda i: (0, i, 0, 0)),
                _full_spec(sew1t.shape), _full_spec(seb1c.shape),
                _full_spec(sew2t.shape), _full_spec(seb2c.shape),
                _full_spec(w0te.shape), _full_spec(w0to.shape),
                _full_spec(b0c.shape)]
    for wt, bc in ws:
        args += [wt, bc]
        in_specs += [_full_spec(wt.shape), _full_spec(bc.shape)]
    args += [wot, bo]
    in_specs += [_full_spec(wot.shape), _full_spec(bo.shape)]
    return pl.pallas_call(
        _tc_dense_body,
        grid=(_B // _BLK,),
        in_specs=in_specs,
        out_specs=pl.BlockSpec((1, _BLK), lambda i: (0, i)),
        out_shape=jax.ShapeDtypeStruct((1, _B), jnp.float32),
    )(*args)


def kernel(f0, f1, f2, f3, f4, f5, f6, f7, f8, f9, f10, f11, f12, f13, f14,
           f15, f16, f17, f18, f19, f20, f21, f22, f23, f24, f25,
           emb, se_w1, se_b1, se_w2, se_b2,
           w0, b0, g0, be0, w1, b1, g1, be1, w2, b2, g2, be2,
           w3, b3, g3, be3, wo, bo):
    fs = jnp.stack([f0, f1, f2, f3, f4, f5, f6, f7, f8, f9, f10, f11, f12,
                    f13, f14, f15, f16, f17, f18, f19, f20, f21, f22, f23,
                    f24, f25], axis=0)
    idxg = fs.reshape(_NW, _CPW * _CHUNK)
    tableT = jnp.transpose(emb, (0, 2, 1))

    tpk = _repack(tableT)
    tpk2 = tpk.reshape(_NF * _VPK * _NSLOT, _HD)

    xi4 = _sc_gather(tpk2, idxg)

    s = 1.0 / jnp.sqrt(jnp.float32(1.0 + _EPS))
    folded = []
    for w, b, g, be in ((w0, b0, g0, be0), (w1, b1, g1, be1),
                        (w2, b2, g2, be2), (w3, b3, g3, be3)):
        gs = g * s
        folded.append((w * gs[None, :], (b * gs + be)[:, None]))
    w0f, b0c = folded[0]
    w0r = w0f.reshape(_NF, _D, -1)
    w0te = w0r[:, 0::2, :].reshape(_HTOT, -1).T
    w0to = w0r[:, 1::2, :].reshape(_HTOT, -1).T
    ws = [(wt.T, bc) for wt, bc in folded[1:]]

    out = _tc_dense(xi4, se_w1.T, se_b1[:, None], se_w2.T, se_b2[:, None],
                    w0te, w0to, b0c, ws, wo.T, bo[:, None])
    return out[0]


# speedup vs baseline: 43.7057x; 1.0284x over previous
import functools

import jax
import jax.numpy as jnp
from jax import lax
from jax.experimental import pallas as pl
from jax.experimental.pallas import tpu as pltpu
from jax.experimental.pallas import tpu_sc as plsc

_NF = 26
_B = 4096
_V = 100001
_D = 32
_HD = _D // 2
_TOT = _NF * _D
_EPS = 1e-5

_VPK = 12544
_NSLOT = 8
_VG = 2
_VROWS = _VPK // _VG

_NC = 2
_NS = 16
_NW = _NC * _NS
_CHUNK = 128
_BPC = _B // _CHUNK
_CPW = 13


def _repack_body(*refs):
    out_ref = refs[-1]
    ki = lax.broadcasted_iota(jnp.int32, (_NSLOT * _D, 128), 0)
    ci = lax.broadcasted_iota(jnp.int32, (_NSLOT * _D, 128), 1)
    src_even = (ci // _HD) * _D + 2 * lax.rem(ci, _HD)
    sel_e = jnp.where(ki == src_even, 1.0, 0.0).astype(jnp.bfloat16)
    sel_o = jnp.where(ki == src_even + 1, 1.0, 0.0).astype(jnp.bfloat16)

    x8 = jnp.concatenate([r[0] for r in refs[:-1]],
                         axis=0).astype(jnp.bfloat16)
    dims = (((0,), (0,)), ((), ()))
    t_e = lax.dot_general(x8, sel_e, dims,
                          preferred_element_type=jnp.float32)
    t_o = lax.dot_general(x8, sel_o, dims,
                          preferred_element_type=jnp.float32)
    be = lax.bitcast_convert_type(t_e, jnp.int32)
    bo = lax.bitcast_convert_type(t_o, jnp.int32)
    out_ref[0] = jnp.bitwise_or(
        lax.shift_right_logical(be, 16),
        jnp.bitwise_and(bo, jnp.int32(-65536)))


_NFH = _NF // 2


def _repack(tableT, f0):
    def spec(s):
        return pl.BlockSpec((1, _D, _VROWS),
                            lambda f, h, s=s: (f0 + f, 0, s * _VG + h))
    return pl.pallas_call(
        _repack_body,
        grid=(_NFH, _VG),
        in_specs=[spec(s) for s in range(_NSLOT)],
        out_specs=pl.BlockSpec((1, _VROWS, 128), lambda f, h: (f, h, 0)),
        out_shape=jax.ShapeDtypeStruct((_NFH, _VPK, 128), jnp.int32),
        compiler_params=pltpu.CompilerParams(
            fuse_transposed_lhs_in_matmul=True),
    )(*([tableT] * _NSLOT))


def _sc_gather_body(tpk, idxg, out, idx_v, pidx_v, pk_a, pk_b, slab_v, sem_a,
                    sem_b):
    wid = lax.axis_index("s") * _NC + lax.axis_index("c")
    pltpu.sync_copy(idxg.at[wid], idx_v)

    def pidx_chunk(c, carry):
        g = wid * _CPW + c
        f = g // _BPC
        for jg in range(_CHUNK // 16):
            sl = pl.ds(c * _CHUNK + jg * 16, 16)
            v = idx_v[sl]
            pidx_v[sl] = ((f * _VPK + lax.rem(v, _VPK)) * _NSLOT
                          + v // _VPK)
        return carry

    lax.fori_loop(0, _CPW, pidx_chunk, 0)

    def fire(c, pk, sem):
        pltpu.async_copy(tpk.at[pidx_v.at[pl.ds(c * _CHUNK, _CHUNK)]], pk,
                         sem)

    def wait(pk, sem):
        pltpu.make_async_copy(tpk.at[pl.ds(0, _CHUNK)], pk, sem).wait()

    def extract(c, pk):
        g = wid * _CPW + c
        f = g // _BPC
        bb = lax.rem(g, _BPC)
        lanes = lax.iota(jnp.int32, 16)
        for jg in range(_CHUNK // 16):
            for k in range(16):
                j = jg * 16 + k
                plsc.store_scatter(slab_v,
                                   [lanes, jnp.full((16,), j, jnp.int32)],
                                   pk[j])
        pltpu.sync_copy(slab_v, out.at[f, bb])

    fire(0, pk_a, sem_a)

    def pipe(cc, carry):
        c0 = cc * 2
        fire(c0 + 1, pk_b, sem_b)
        wait(pk_a, sem_a)
        extract(c0, pk_a)
        fire(c0 + 2, pk_a, sem_a)
        wait(pk_b, sem_b)
        extract(c0 + 1, pk_b)
        return carry

    lax.fori_loop(0, _CPW // 2, pipe, 0)
    wait(pk_a, sem_a)
    extract(_CPW - 1, pk_a)


@functools.partial(
    pl.kernel,
    out_type=jax.ShapeDtypeStruct((_NFH, _BPC, _HD, _CHUNK), jnp.int32),
    mesh=plsc.VectorSubcoreMesh(
        core_axis_name="c", subcore_axis_name="s",
        num_cores=_NC, num_subcores=_NS,
    ),
    scratch_types=[
        pltpu.VMEM((_CPW * _CHUNK,), jnp.int32),
        pltpu.VMEM((_CPW * _CHUNK,), jnp.int32),
        pltpu.VMEM((_CHUNK, _HD), jnp.int32),
        pltpu.VMEM((_CHUNK, _HD), jnp.int32),
        pltpu.VMEM((_HD, _CHUNK), jnp.int32),
        pltpu.SemaphoreType.DMA,
        pltpu.SemaphoreType.DMA,
    ],
    compiler_params=pltpu.CompilerParams(use_tc_tiling_on_sc=False,
                                         needs_layout_passes=False),
)
def _sc_gather(tpk, idxg, out, idx_v, pidx_v, pk_a, pk_b, slab_v, sem_a,
               sem_b):
    _sc_gather_body(tpk, idxg, out, idx_v, pidx_v, pk_a, pk_b, slab_v,
                    sem_a, sem_b)


_BLK = 512
_HTOT = _NF * _HD


def _sigmoid(x):
    return 1.0 / (1.0 + jnp.exp(-x))


def _tc_dense_body(xi_ref, sew1t_ref, seb1c_ref, sew2t_ref, seb2c_ref,
                   w0te_ref, w0to_ref, b0c_ref, w1t_ref, b1c_ref,
                   w2t_ref, b2c_ref, w3t_ref, b3c_ref, wot_ref, bo_ref,
                   out_ref):
    w4 = xi_ref[...]
    wi = jnp.transpose(w4, (0, 2, 1, 3)).reshape(_HTOT, _BLK)
    xe = lax.bitcast_convert_type(lax.shift_left(wi, 16), jnp.float32)
    xo = lax.bitcast_convert_type(
        jnp.bitwise_and(wi, jnp.int32(-65536)), jnp.float32)

    ri = lax.broadcasted_iota(jnp.int32, (_NF, _HTOT), 0)
    ci = lax.broadcasted_iota(jnp.int32, (_NF, _HTOT), 1) // _HD
    mT = jnp.where(ri == ci, 1.0 / _D, 0.0)
    sqT = (jnp.dot(mT, xe, preferred_element_type=jnp.float32)
           + jnp.dot(mT, xo, preferred_element_type=jnp.float32))

    hT = jnp.maximum(
        jnp.dot(sew1t_ref[...], sqT, preferred_element_type=jnp.float32)
        + seb1c_ref[...], 0.0)
    wseT = _sigmoid(
        jnp.dot(sew2t_ref[...], hT, preferred_element_type=jnp.float32)
        + seb2c_ref[...])

    r2 = lax.broadcasted_iota(jnp.int32, (_HTOT, _NF), 0) // _HD
    c2 = lax.broadcasted_iota(jnp.int32, (_HTOT, _NF), 1)
    eT = jnp.where(r2 == c2, 1.0, 0.0)
    scale = jnp.dot(eT, wseT, preferred_element_type=jnp.float32)
    xe = xe * scale
    xo = xo * scale

    x = jnp.maximum(
        jnp.dot(w0te_ref[...], xe, preferred_element_type=jnp.float32)
        + jnp.dot(w0to_ref[...], xo, preferred_element_type=jnp.float32)
        + b0c_ref[...], 0.0)
    for wt_ref, bc_ref in ((w1t_ref, b1c_ref), (w2t_ref, b2c_ref),
                           (w3t_ref, b3c_ref)):
        x = jnp.maximum(
            jnp.dot(wt_ref[...], x, preferred_element_type=jnp.float32)
            + bc_ref[...], 0.0)

    out_ref[...] = (jnp.dot(wot_ref[...], x,
                            preferred_element_type=jnp.float32) + bo_ref[...])


def _full_spec(shape):
    return pl.BlockSpec(shape, lambda i: tuple(0 for _ in shape))


def _tc_dense(xi4, sew1t, seb1c, sew2t, seb2c, w0te, w0to, b0c, ws, wot, bo):
    args = [xi4, sew1t, seb1c, sew2t, seb2c, w0te, w0to, b0c]
    in_specs = [pl.BlockSpec((_NF, _BLK // _CHUNK, _HD, _CHUNK),
                             lambda i: (0, i, 0, 0)),
                _full_spec(sew1t.shape), _full_spec(seb1c.shape),
                _full_spec(sew2t.shape), _full_spec(seb2c.shape),
                _full_spec(w0te.shape), _full_spec(w0to.shape),
                _full_spec(b0c.shape)]
    for wt, bc in ws:
        args += [wt, bc]
        in_specs += [_full_spec(wt.shape), _full_spec(bc.shape)]
    args += [wot, bo]
    in_specs += [_full_spec(wot.shape), _full_spec(bo.shape)]
    return pl.pallas_call(
        _tc_dense_body,
        grid=(_B // _BLK,),
        in_specs=in_specs,
        out_specs=pl.BlockSpec((1, _BLK), lambda i: (0, i)),
        out_shape=jax.ShapeDtypeStruct((1, _B), jnp.float32),
    )(*args)


def kernel(f0, f1, f2, f3, f4, f5, f6, f7, f8, f9, f10, f11, f12, f13, f14,
           f15, f16, f17, f18, f19, f20, f21, f22, f23, f24, f25,
           emb, se_w1, se_b1, se_w2, se_b2,
           w0, b0, g0, be0, w1, b1, g1, be1, w2, b2, g2, be2,
           w3, b3, g3, be3, wo, bo):
    fs = jnp.stack([f0, f1, f2, f3, f4, f5, f6, f7, f8, f9, f10, f11, f12,
                    f13, f14, f15, f16, f17, f18, f19, f20, f21, f22, f23,
                    f24, f25], axis=0)
    tableT = jnp.transpose(emb, (0, 2, 1))

    halves = []
    for f0 in (0, _NFH):
        idxh = fs[f0:f0 + _NFH].reshape(_NW, _CPW * _CHUNK)
        tpk = _repack(tableT, f0)
        tpk2 = tpk.reshape(_NFH * _VPK * _NSLOT, _HD)
        halves.append(_sc_gather(tpk2, idxh))
    xi4 = jnp.concatenate(halves, axis=0)

    s = 1.0 / jnp.sqrt(jnp.float32(1.0 + _EPS))
    folded = []
    for w, b, g, be in ((w0, b0, g0, be0), (w1, b1, g1, be1),
                        (w2, b2, g2, be2), (w3, b3, g3, be3)):
        gs = g * s
        folded.append((w * gs[None, :], (b * gs + be)[:, None]))
    w0f, b0c = folded[0]
    w0r = w0f.reshape(_NF, _D, -1)
    w0te = w0r[:, 0::2, :].reshape(_HTOT, -1).T
    w0to = w0r[:, 1::2, :].reshape(_HTOT, -1).T
    ws = [(wt.T, bc) for wt, bc in folded[1:]]

    out = _tc_dense(xi4, se_w1.T, se_b1[:, None], se_w2.T, se_b2[:, None],
                    w0te, w0to, b0c, ws, wo.T, bo[:, None])
    return out[0]


# speedup vs baseline: 43.7370x; 1.0007x over previous
import functools

import jax
import jax.numpy as jnp
from jax import lax
from jax.experimental import pallas as pl
from jax.experimental.pallas import tpu as pltpu
from jax.experimental.pallas import tpu_sc as plsc

_NF = 26
_B = 4096
_V = 100001
_D = 32
_HD = _D // 2
_TOT = _NF * _D
_EPS = 1e-5

_VPK = 12544
_NSLOT = 8
_VG = 2
_VROWS = _VPK // _VG

_NC = 2
_NS = 16
_NW = _NC * _NS
_CHUNK = 128
_BPC = _B // _CHUNK
_CPW = 13


def _repack_body(*refs):
    out_ref = refs[-1]
    ki = lax.broadcasted_iota(jnp.int32, (_NSLOT * _D, 128), 0)
    ci = lax.broadcasted_iota(jnp.int32, (_NSLOT * _D, 128), 1)
    src_even = (ci // _HD) * _D + 2 * lax.rem(ci, _HD)
    sel_e = jnp.where(ki == src_even, 1.0, 0.0).astype(jnp.bfloat16)
    sel_o = jnp.where(ki == src_even + 1, 1.0, 0.0).astype(jnp.bfloat16)

    x8 = jnp.concatenate([r[0] for r in refs[:-1]],
                         axis=0).astype(jnp.bfloat16)
    dims = (((0,), (0,)), ((), ()))
    t_e = lax.dot_general(x8, sel_e, dims,
                          preferred_element_type=jnp.float32)
    t_o = lax.dot_general(x8, sel_o, dims,
                          preferred_element_type=jnp.float32)
    be = lax.bitcast_convert_type(t_e, jnp.int32)
    bo = lax.bitcast_convert_type(t_o, jnp.int32)
    out_ref[0] = jnp.bitwise_or(
        lax.shift_right_logical(be, 16),
        jnp.bitwise_and(bo, jnp.int32(-65536)))


_NFH = _NF // 2


def _repack(tableT, f0):
    def spec(s):
        return pl.BlockSpec((1, _D, _VROWS),
                            lambda f, h, s=s: (f0 + f, 0, s * _VG + h))
    return pl.pallas_call(
        _repack_body,
        grid=(_NFH, _VG),
        in_specs=[spec(s) for s in range(_NSLOT)],
        out_specs=pl.BlockSpec((1, _VROWS, 128), lambda f, h: (f, h, 0)),
        out_shape=jax.ShapeDtypeStruct((_NFH, _VPK, 128), jnp.int32),
        compiler_params=pltpu.CompilerParams(
            dimension_semantics=("parallel", "parallel"),
            fuse_transposed_lhs_in_matmul=True),
    )(*([tableT] * _NSLOT))


def _sc_gather_body(tpk, idxg, out, idx_v, pidx_v, pk_a, pk_b, slab_v, sem_a,
                    sem_b):
    wid = lax.axis_index("s") * _NC + lax.axis_index("c")
    pltpu.sync_copy(idxg.at[wid], idx_v)

    def pidx_chunk(c, carry):
        g = wid * _CPW + c
        f = g // _BPC
        for jg in range(_CHUNK // 16):
            sl = pl.ds(c * _CHUNK + jg * 16, 16)
            v = idx_v[sl]
            pidx_v[sl] = ((f * _VPK + lax.rem(v, _VPK)) * _NSLOT
                          + v // _VPK)
        return carry

    lax.fori_loop(0, _CPW, pidx_chunk, 0)

    def fire(c, pk, sem):
        pltpu.async_copy(tpk.at[pidx_v.at[pl.ds(c * _CHUNK, _CHUNK)]], pk,
                         sem)

    def wait(pk, sem):
        pltpu.make_async_copy(tpk.at[pl.ds(0, _CHUNK)], pk, sem).wait()

    def extract(c, pk):
        g = wid * _CPW + c
        f = g // _BPC
        bb = lax.rem(g, _BPC)
        lanes = lax.iota(jnp.int32, 16)
        for jg in range(_CHUNK // 16):
            for k in range(16):
                j = jg * 16 + k
                plsc.store_scatter(slab_v,
                                   [lanes, jnp.full((16,), j, jnp.int32)],
                                   pk[j])
        pltpu.sync_copy(slab_v, out.at[f, bb])

    fire(0, pk_a, sem_a)

    def pipe(cc, carry):
        c0 = cc * 2
        fire(c0 + 1, pk_b, sem_b)
        wait(pk_a, sem_a)
        extract(c0, pk_a)
        fire(c0 + 2, pk_a, sem_a)
        wait(pk_b, sem_b)
        extract(c0 + 1, pk_b)
        return carry

    lax.fori_loop(0, _CPW // 2, pipe, 0)
    wait(pk_a, sem_a)
    extract(_CPW - 1, pk_a)


@functools.partial(
    pl.kernel,
    out_type=jax.ShapeDtypeStruct((_NFH, _BPC, _HD, _CHUNK), jnp.int32),
    mesh=plsc.VectorSubcoreMesh(
        core_axis_name="c", subcore_axis_name="s",
        num_cores=_NC, num_subcores=_NS,
    ),
    scratch_types=[
        pltpu.VMEM((_CPW * _CHUNK,), jnp.int32),
        pltpu.VMEM((_CPW * _CHUNK,), jnp.int32),
        pltpu.VMEM((_CHUNK, _HD), jnp.int32),
        pltpu.VMEM((_CHUNK, _HD), jnp.int32),
        pltpu.VMEM((_HD, _CHUNK), jnp.int32),
        pltpu.SemaphoreType.DMA,
        pltpu.SemaphoreType.DMA,
    ],
    compiler_params=pltpu.CompilerParams(use_tc_tiling_on_sc=False,
                                         needs_layout_passes=False),
)
def _sc_gather(tpk, idxg, out, idx_v, pidx_v, pk_a, pk_b, slab_v, sem_a,
               sem_b):
    _sc_gather_body(tpk, idxg, out, idx_v, pidx_v, pk_a, pk_b, slab_v,
                    sem_a, sem_b)


_BLK = 512
_HTOT = _NF * _HD


def _sigmoid(x):
    return 1.0 / (1.0 + jnp.exp(-x))


def _tc_dense_body(xi_ref, sew1t_ref, seb1c_ref, sew2t_ref, seb2c_ref,
                   w0te_ref, w0to_ref, b0c_ref, w1t_ref, b1c_ref,
                   w2t_ref, b2c_ref, w3t_ref, b3c_ref, wot_ref, bo_ref,
                   out_ref):
    w4 = xi_ref[...]
    wi = jnp.transpose(w4, (0, 2, 1, 3)).reshape(_HTOT, _BLK)
    xe = lax.bitcast_convert_type(lax.shift_left(wi, 16), jnp.float32)
    xo = lax.bitcast_convert_type(
        jnp.bitwise_and(wi, jnp.int32(-65536)), jnp.float32)

    ri = lax.broadcasted_iota(jnp.int32, (_NF, _HTOT), 0)
    ci = lax.broadcasted_iota(jnp.int32, (_NF, _HTOT), 1) // _HD
    mT = jnp.where(ri == ci, 1.0 / _D, 0.0)
    sqT = (jnp.dot(mT, xe, preferred_element_type=jnp.float32)
           + jnp.dot(mT, xo, preferred_element_type=jnp.float32))

    hT = jnp.maximum(
        jnp.dot(sew1t_ref[...], sqT, preferred_element_type=jnp.float32)
        + seb1c_ref[...], 0.0)
    wseT = _sigmoid(
        jnp.dot(sew2t_ref[...], hT, preferred_element_type=jnp.float32)
        + seb2c_ref[...])

    r2 = lax.broadcasted_iota(jnp.int32, (_HTOT, _NF), 0) // _HD
    c2 = lax.broadcasted_iota(jnp.int32, (_HTOT, _NF), 1)
    eT = jnp.where(r2 == c2, 1.0, 0.0)
    scale = jnp.dot(eT, wseT, preferred_element_type=jnp.float32)
    xe = xe * scale
    xo = xo * scale

    x = jnp.maximum(
        jnp.dot(w0te_ref[...], xe, preferred_element_type=jnp.float32)
        + jnp.dot(w0to_ref[...], xo, preferred_element_type=jnp.float32)
        + b0c_ref[...], 0.0)
    for wt_ref, bc_ref in ((w1t_ref, b1c_ref), (w2t_ref, b2c_ref),
                           (w3t_ref, b3c_ref)):
        x = jnp.maximum(
            jnp.dot(wt_ref[...], x, preferred_element_type=jnp.float32)
            + bc_ref[...], 0.0)

    out_ref[...] = (jnp.dot(wot_ref[...], x,
                            preferred_element_type=jnp.float32) + bo_ref[...])


def _full_spec(shape):
    return pl.BlockSpec(shape, lambda i: tuple(0 for _ in shape))


def _tc_dense(xi4, sew1t, seb1c, sew2t, seb2c, w0te, w0to, b0c, ws, wot, bo):
    args = [xi4, sew1t, seb1c, sew2t, seb2c, w0te, w0to, b0c]
    in_specs = [pl.BlockSpec((_NF, _BLK // _CHUNK, _HD, _CHUNK),
                             lambda i: (0, i, 0, 0)),
                _full_spec(sew1t.shape), _full_spec(seb1c.shape),
                _full_spec(sew2t.shape), _full_spec(seb2c.shape),
                _full_spec(w0te.shape), _full_spec(w0to.shape),
                _full_spec(b0c.shape)]
    for wt, bc in ws:
        args += [wt, bc]
        in_specs += [_full_spec(wt.shape), _full_spec(bc.shape)]
    args += [wot, bo]
    in_specs += [_full_spec(wot.shape), _full_spec(bo.shape)]
    return pl.pallas_call(
        _tc_dense_body,
        grid=(_B // _BLK,),
        in_specs=in_specs,
        out_specs=pl.BlockSpec((1, _BLK), lambda i: (0, i)),
        out_shape=jax.ShapeDtypeStruct((1, _B), jnp.float32),
    )(*args)


def kernel(f0, f1, f2, f3, f4, f5, f6, f7, f8, f9, f10, f11, f12, f13, f14,
           f15, f16, f17, f18, f19, f20, f21, f22, f23, f24, f25,
           emb, se_w1, se_b1, se_w2, se_b2,
           w0, b0, g0, be0, w1, b1, g1, be1, w2, b2, g2, be2,
           w3, b3, g3, be3, wo, bo):
    fs = jnp.stack([f0, f1, f2, f3, f4, f5, f6, f7, f8, f9, f10, f11, f12,
                    f13, f14, f15, f16, f17, f18, f19, f20, f21, f22, f23,
                    f24, f25], axis=0)
    tableT = jnp.transpose(emb, (0, 2, 1))

    halves = []
    for f0 in (0, _NFH):
        idxh = fs[f0:f0 + _NFH].reshape(_NW, _CPW * _CHUNK)
        tpk = _repack(tableT, f0)
        tpk2 = tpk.reshape(_NFH * _VPK * _NSLOT, _HD)
        halves.append(_sc_gather(tpk2, idxh))
    xi4 = jnp.concatenate(halves, axis=0)

    s = 1.0 / jnp.sqrt(jnp.float32(1.0 + _EPS))
    folded = []
    for w, b, g, be in ((w0, b0, g0, be0), (w1, b1, g1, be1),
                        (w2, b2, g2, be2), (w3, b3, g3, be3)):
        gs = g * s
        folded.append((w * gs[None, :], (b * gs + be)[:, None]))
    w0f, b0c = folded[0]
    w0r = w0f.reshape(_NF, _D, -1)
    w0te = w0r[:, 0::2, :].reshape(_HTOT, -1).T
    w0to = w0r[:, 1::2, :].reshape(_HTOT, -1).T
    ws = [(wt.T, bc) for wt, bc in folded[1:]]

    out = _tc_dense(xi4, se_w1.T, se_b1[:, None], se_w2.T, se_b2[:, None],
                    w0te, w0to, b0c, ws, wo.T, bo[:, None])
    return out[0]
